# baseline (speedup 1.0000x reference)
import jax
import jax.numpy as jnp
from jax import lax
from jax.experimental import pallas as pl
from jax.experimental.pallas import tpu as pltpu
from jax.experimental.pallas import tpu_sc as plsc

VOCAB = 1000
NC, NS = 2, 16
NW = NC * NS


def _sc_body(x_hbm, z_hbm, out_hbm, idx_v, buf0, buf1, sem0, sem1):
    n_batch = out_hbm.shape[0]
    nf = out_hbm.shape[1]
    bpw = n_batch // NW
    wid = lax.axis_index("s") * NC + lax.axis_index("c")
    base_b = wid * bpw

    pltpu.sync_copy(x_hbm.at[pl.ds(base_b * nf, bpw * nf)],
                    idx_v.at[pl.ds(0, bpw * nf)])
    pltpu.async_copy(z_hbm, buf0, sem0).wait()
    pltpu.async_copy(z_hbm, buf1, sem1).wait()

    lane = lax.iota(jnp.int32, 16)
    zero16 = jnp.zeros((16,), jnp.int32)
    ones = jnp.full((16,), 1.0, jnp.float32)
    zf = jnp.zeros((16,), jnp.float32)
    mask_hi = lane < (nf - 16)

    def scatter(buf, c, val):
        col_lo = plsc.load_gather(idx_v, [lane + c * nf])
        col_hi = plsc.load_gather(idx_v, [lane + (c * nf + 16)])
        plsc.store_scatter(buf, [zero16, lane, col_lo], val)
        plsc.store_scatter(buf, [zero16, lane + 16, col_hi], val,
                           mask=mask_hi)

    def dma(buf, sem, c):
        return pltpu.make_async_copy(
            buf, out_hbm.at[pl.ds(base_b + c, 1)], sem)

    scatter(buf0, 0, ones)
    dma(buf0, sem0, 0).start()
    scatter(buf1, 1, ones)
    dma(buf1, sem1, 1).start()

    def pair(p, carry):
        for b, (buf, sem) in enumerate(((buf0, sem0), (buf1, sem1))):
            c_prev = 2 * p - 2 + b
            c_new = 2 * p + b
            dma(buf, sem, c_prev).wait()
            scatter(buf, c_prev, zf)
            scatter(buf, c_new, ones)
            dma(buf, sem, c_new).start()
        return carry

    lax.fori_loop(1, bpw // 2, pair, 0)
    dma(buf0, sem0, bpw - 2).wait()
    dma(buf1, sem1, bpw - 1).wait()


def kernel(x):
    b, f = x.shape
    n = b * f
    call = pl.kernel(
        _sc_body,
        out_type=jax.ShapeDtypeStruct((b, f, VOCAB), jnp.float32),
        mesh=plsc.VectorSubcoreMesh(
            core_axis_name="c", subcore_axis_name="s",
            num_cores=NC, num_subcores=NS),
        scratch_types=[
            pltpu.VMEM((n // NW + 16,), jnp.int32),
            pltpu.VMEM((1, f, VOCAB), jnp.float32),
            pltpu.VMEM((1, f, VOCAB), jnp.float32),
            pltpu.SemaphoreType.DMA,
            pltpu.SemaphoreType.DMA,
        ],
        compiler_params=pltpu.CompilerParams(
            needs_layout_passes=False, use_tc_tiling_on_sc=True),
    )
    return call(x.reshape(n), jnp.zeros((1, f, VOCAB), jnp.float32))
